# baseline (speedup 1.0000x reference)
import functools

import jax
import jax.numpy as jnp
from jax import lax
from jax.experimental import pallas as pl
from jax.experimental.pallas import tpu as pltpu
from jax.experimental.pallas import tpu_sc as plsc

D = 64
NC = 2
NS = 16
NW = NC * NS
IPS = 128
K = 4
CHUNK = K * IPS


@functools.partial(jax.jit, static_argnums=(2,))
def _emb(words_2d, table, b_total):
    b_per_w = b_total // NW
    nchunks = b_per_w // CHUNK
    rows_per_w = b_per_w // IPS

    @functools.partial(
        pl.kernel,
        mesh=plsc.VectorSubcoreMesh(core_axis_name="c", subcore_axis_name="s"),
        out_type=jax.ShapeDtypeStruct((b_total, D), jnp.float32),
        scratch_types=[
            pltpu.VMEM((K, IPS), jnp.int32),
            pltpu.VMEM((CHUNK, D), jnp.float32),
            pltpu.SemaphoreType.DMA,
        ],
        compiler_params=pltpu.CompilerParams(use_tc_tiling_on_sc=False),
    )
    def k(words_hbm, table_hbm, out_hbm, idx_v, rows_v, sem):
        wid = lax.axis_index("s") * NC + lax.axis_index("c")
        base = wid * b_per_w

        base_row = wid * rows_per_w

        def body(g, carry):
            off = base + g * CHUNK
            pltpu.sync_copy(words_hbm.at[pl.ds(base_row + g * K, K)], idx_v)
            copies = [
                pltpu.async_copy(
                    table_hbm.at[idx_v.at[j]],
                    rows_v.at[pl.ds(j * IPS, IPS)],
                    sem,
                )
                for j in range(K)
            ]
            for c in copies:
                c.wait()
            pltpu.sync_copy(rows_v, out_hbm.at[pl.ds(off, CHUNK)])
            return carry

        lax.fori_loop(0, nchunks, body, 0)

    return k(words_2d, table)


def kernel(words, table):
    b, h = words.shape
    flat = words.reshape((b * h) // IPS, IPS)
    out = _emb(flat, table, b * h)
    return out.reshape(b, h, D)


# speedup vs baseline: 1.0276x; 1.0276x over previous
import functools

import jax
import jax.numpy as jnp
from jax import lax
from jax.experimental import pallas as pl
from jax.experimental.pallas import tpu as pltpu
from jax.experimental.pallas import tpu_sc as plsc

D = 64
NC = 2
NS = 16
NW = NC * NS
IPS = 128
K = 2
CHUNK = K * IPS
NBUF = 4


@functools.partial(jax.jit, static_argnums=(2,))
def _emb(words_2d, table, b_total):
    b_per_w = b_total // NW
    nchunks = b_per_w // CHUNK
    rows_per_w = b_per_w // IPS
    nsteps = nchunks // NBUF

    @functools.partial(
        pl.kernel,
        mesh=plsc.VectorSubcoreMesh(core_axis_name="c", subcore_axis_name="s"),
        out_type=jax.ShapeDtypeStruct((b_total, D), jnp.float32),
        scratch_types=[
            pltpu.VMEM((NBUF, K, IPS), jnp.int32),
            pltpu.VMEM((NBUF, CHUNK, D), jnp.float32),
            pltpu.SemaphoreType.DMA((NBUF,)),
            pltpu.SemaphoreType.DMA((NBUF,)),
        ],
        compiler_params=pltpu.CompilerParams(use_tc_tiling_on_sc=False),
    )
    def k(words_hbm, table_hbm, out_hbm, idx_v, rows_v, gsem, osem):
        wid = lax.axis_index("s") * NC + lax.axis_index("c")
        base = wid * b_per_w
        base_row = wid * rows_per_w

        def load_and_fire(g, b):
            pltpu.sync_copy(
                words_hbm.at[pl.ds(base_row + g * K, K)], idx_v.at[b]
            )
            for j in range(K):
                pltpu.async_copy(
                    table_hbm.at[idx_v.at[b, j]],
                    rows_v.at[b, pl.ds(j * IPS, IPS)],
                    gsem.at[b],
                )

        def wait_gathers(b):
            for j in range(K):
                pltpu.make_async_copy(
                    table_hbm.at[idx_v.at[b, j]],
                    rows_v.at[b, pl.ds(j * IPS, IPS)],
                    gsem.at[b],
                ).wait()

        def out_copy(g, b):
            return pltpu.make_async_copy(
                rows_v.at[b],
                out_hbm.at[pl.ds(base + g * CHUNK, CHUNK)],
                osem.at[b],
            )

        for b in range(NBUF):
            load_and_fire(b, b)

        def step(t, carry):
            for b in range(NBUF):
                g = t * NBUF + b
                wait_gathers(b)
                out_copy(g, b).start()
                out_copy(g, b).wait()
                load_and_fire(g + NBUF, b)
            return carry

        lax.fori_loop(0, nsteps - 1, step, 0)

        for b in range(NBUF):
            g = (nsteps - 1) * NBUF + b
            wait_gathers(b)
            out_copy(g, b).start()
        for b in range(NBUF):
            g = (nsteps - 1) * NBUF + b
            out_copy(g, b).wait()

    return k(words_2d, table)


def kernel(words, table):
    b, h = words.shape
    flat = words.reshape((b * h) // IPS, IPS)
    out = _emb(flat, table, b * h)
    return out.reshape(b, h, D)


# speedup vs baseline: 1.0550x; 1.0267x over previous
import functools

import jax
import jax.numpy as jnp
from jax import lax
from jax.experimental import pallas as pl
from jax.experimental.pallas import tpu as pltpu
from jax.experimental.pallas import tpu_sc as plsc

D = 64
NC = 2
NS = 16
NW = NC * NS
IPS = 128
K = 2
CHUNK = K * IPS
NBUF = 4


@functools.partial(jax.jit, static_argnums=(2,))
def _emb(words_flat, table_flat, b_total):
    words_2d = words_flat.reshape(b_total // IPS, IPS)
    table = table_flat.reshape(table_flat.shape[0] // D, D)
    b_per_w = b_total // NW
    nchunks = b_per_w // CHUNK
    rows_per_w = b_per_w // IPS
    nsteps = nchunks // NBUF

    @functools.partial(
        pl.kernel,
        mesh=plsc.VectorSubcoreMesh(core_axis_name="c", subcore_axis_name="s"),
        out_type=jax.ShapeDtypeStruct((b_total, D), jnp.float32),
        scratch_types=[
            pltpu.VMEM((NBUF, K, IPS), jnp.int32),
            pltpu.VMEM((NBUF, CHUNK, D), jnp.float32),
            pltpu.SemaphoreType.DMA((NBUF,)),
            pltpu.SemaphoreType.DMA((NBUF,)),
        ],
        compiler_params=pltpu.CompilerParams(use_tc_tiling_on_sc=False),
    )
    def k(words_hbm, table_hbm, out_hbm, idx_v, rows_v, gsem, osem):
        wid = lax.axis_index("s") * NC + lax.axis_index("c")
        base = wid * b_per_w
        base_row = wid * rows_per_w

        def load_and_fire(g, b):
            pltpu.sync_copy(
                words_hbm.at[pl.ds(base_row + g * K, K)], idx_v.at[b]
            )
            for j in range(K):
                pltpu.async_copy(
                    table_hbm.at[idx_v.at[b, j]],
                    rows_v.at[b, pl.ds(j * IPS, IPS)],
                    gsem.at[b],
                )

        def wait_gathers(b):
            for j in range(K):
                pltpu.make_async_copy(
                    table_hbm.at[idx_v.at[b, j]],
                    rows_v.at[b, pl.ds(j * IPS, IPS)],
                    gsem.at[b],
                ).wait()

        def out_copy(g, b):
            return pltpu.make_async_copy(
                rows_v.at[b],
                out_hbm.at[pl.ds(base + g * CHUNK, CHUNK)],
                osem.at[b],
            )

        for b in range(NBUF):
            load_and_fire(b, b)

        def step(t, carry):
            for b in range(NBUF):
                g = t * NBUF + b
                wait_gathers(b)
                out_copy(g, b).start()
                out_copy(g, b).wait()
                load_and_fire(g + NBUF, b)
            return carry

        lax.fori_loop(0, nsteps - 1, step, 0)

        for b in range(NBUF):
            g = (nsteps - 1) * NBUF + b
            wait_gathers(b)
            out_copy(g, b).start()
        for b in range(NBUF):
            g = (nsteps - 1) * NBUF + b
            out_copy(g, b).wait()

    return k(words_2d, table)


def kernel(words, table):
    b, h = words.shape
    wt = words.T.reshape(b * h)
    out = _emb(wt, table.reshape(-1), b * h)
    return out.reshape(h, b, D).transpose(1, 0, 2)
